# Initial kernel scaffold; baseline (speedup 1.0000x reference)
import jax
import jax.numpy as jnp
from jax.experimental import pallas as pl


def kernel(query_emb, edge_index, edge_attr, node_emb, W, att_src, att_dst, W_e, att_edge, bias):
    raise NotImplementedError("write your pallas kernel here")


# speedup vs baseline: 3.5180x; 3.5180x over previous
import functools

import jax
import jax.numpy as jnp
from jax import lax
from jax.experimental import pallas as pl
from jax.experimental.pallas import tpu as pltpu
from jax.experimental.pallas import tpu_sc as plsc

F32 = jnp.float32
HI = lax.Precision.HIGHEST


def _nodes_body(x_ref, w_ref, avs_ref, avd_ref, h_ref, as_ref, ad_ref):
    x = x_ref[...]
    h = jnp.dot(x, w_ref[...], precision=HI)
    h_ref[...] = h
    as_ref[...] = jnp.dot(h, avs_ref[0], precision=HI)[None, None, :]
    ad_ref[...] = jnp.dot(h, avd_ref[0], precision=HI)[None, None, :]


def _nodes_stage(node_emb, W, att_src, att_dst, NB=1000):
    N, H = node_emb.shape
    return pl.pallas_call(
        _nodes_body,
        grid=(N // NB,),
        in_specs=[
            pl.BlockSpec((NB, H), lambda i: (i, 0)),
            pl.BlockSpec((H, H), lambda i: (0, 0)),
            pl.BlockSpec((1, H), lambda i: (0, 0)),
            pl.BlockSpec((1, H), lambda i: (0, 0)),
        ],
        out_specs=[
            pl.BlockSpec((NB, H), lambda i: (i, 0)),
            pl.BlockSpec((1, 1, NB), lambda i: (i, 0, 0)),
            pl.BlockSpec((1, 1, NB), lambda i: (i, 0, 0)),
        ],
        out_shape=[
            jax.ShapeDtypeStruct((N, H), F32),
            jax.ShapeDtypeStruct((N // NB, 1, NB), F32),
            jax.ShapeDtypeStruct((N // NB, 1, NB), F32),
        ],
    )(node_emb, W, att_src.reshape(1, H), att_dst.reshape(1, H))


def _edges_body(ea_ref, we_ref, av_ref, ei_ref, ae_ref, gq_ref):
    wv = jnp.dot(we_ref[...], av_ref[...], precision=HI)
    ae_ref[...] = jnp.dot(ea_ref[...], wv, precision=HI)
    src = ei_ref[0]
    g = src * 4
    gq_ref[...] = jnp.stack([g, g + 1, g + 2, g + 3], axis=0)


def _edge_scal_stage(edge_attr, W_e, att_edge, edge_index, EB=1280):
    E, H = edge_attr.shape
    return pl.pallas_call(
        _edges_body,
        grid=(E // EB,),
        in_specs=[
            pl.BlockSpec((EB, H), lambda i: (i, 0)),
            pl.BlockSpec((H, H), lambda i: (0, 0)),
            pl.BlockSpec((H, 1), lambda i: (0, 0)),
            pl.BlockSpec((2, EB), lambda i: (0, i)),
        ],
        out_specs=[
            pl.BlockSpec((EB, 1), lambda i: (i, 0)),
            pl.BlockSpec((4, EB), lambda i: (0, i)),
        ],
        out_shape=[
            jax.ShapeDtypeStruct((E, 1), F32),
            jax.ShapeDtypeStruct((4, E), jnp.int32),
        ],
    )(edge_attr, W_e, att_edge.reshape(H, 1), edge_index)


def _sc_stage(src, dst, ae, gq, h4, asrc, adst, qbase):
    E = src.shape[0]
    N = asrc.shape[0]
    QW = h4.shape[1]
    WID = QW + 16
    K = 80
    NSUB = 16
    ept = E // NSUB
    nch = ept // K
    nrc = N // K
    nrc_per_tile = (nrc + NSUB - 1) // NSUB

    mesh = plsc.VectorSubcoreMesh(core_axis_name="c", subcore_axis_name="s")

    def body(src_hbm, dst_hbm, ae_hbm, gq_hbm, h4_hbm, asrc_hbm, adst_hbm,
             acc_hbm, asrc_v, adst_v, srcb, dstb, aeb, gixb, pbuf, rowbuf,
             scbuf, acc_sh, sem):
        c = lax.axis_index("c")
        s = lax.axis_index("s")
        q = qbase + c
        pltpu.sync_copy(asrc_hbm, asrc_v)
        pltpu.sync_copy(adst_hbm, adst_v)

        def zrow(r, _):
            for v in range(WID // 16):
                scbuf[r, pl.ds(v * 16, 16)] = jnp.zeros((16,), F32)
            return 0
        lax.fori_loop(0, K, zrow, 0)

        def zchunk(k, _):
            ci = k * NSUB + s
            @pl.when(ci < nrc)
            def _():
                pltpu.sync_copy(scbuf, acc_sh.at[pl.ds(ci * K, K)])
            return 0
        lax.fori_loop(0, nrc_per_tile, zchunk, 0)
        plsc.subcore_barrier()

        lanes = lax.iota(jnp.int32, 16)

        def chunk(k, _):
            base = s * ept + k * K
            pltpu.sync_copy(src_hbm.at[pl.ds(base, K)], srcb)
            pltpu.sync_copy(dst_hbm.at[pl.ds(base, K)], dstb)
            pltpu.sync_copy(ae_hbm.at[pl.ds(base, K)], aeb)
            pltpu.sync_copy(gq_hbm.at[q, pl.ds(base, K)], gixb)
            cp = pltpu.async_copy(h4_hbm.at[gixb], rowbuf, sem)
            for j in range(K // 16):
                sv = srcb[pl.ds(j * 16, 16)]
                dv = dstb[pl.ds(j * 16, 16)]
                a1 = plsc.load_gather(asrc_v, [sv])
                a2 = plsc.load_gather(adst_v, [dv])
                al = a1 + a2 + aeb[pl.ds(j * 16, 16)]
                al = jnp.maximum(al, al * 0.2)
                pbuf[pl.ds(j * 16, 16)] = jnp.exp(al)
            cp.wait()

            def srow(e, _):
                bidx = jnp.full((16,), e, jnp.int32)
                bp = plsc.load_gather(pbuf, [bidx])
                for v in range(QW // 16):
                    scbuf[e, pl.ds(v * 16, 16)] = rowbuf[e, pl.ds(v * 16, 16)] * bp
                bae = plsc.load_gather(aeb, [bidx])
                extra = jnp.where(lanes == 0, bp,
                        jnp.where(lanes == 1, bae,
                        jnp.where(lanes == 2, jnp.float32(1.0), jnp.float32(0.0))))
                scbuf[e, pl.ds(QW, 16)] = extra
                return 0
            lax.fori_loop(0, K, srow, 0)
            pltpu.sync_copy(scbuf, acc_sh.at[dstb], add=True)
            return 0
        lax.fori_loop(0, nch, chunk, 0)
        plsc.subcore_barrier()

        def dump(k, _):
            ci = k * NSUB + s
            @pl.when(ci < nrc)
            def _():
                pltpu.sync_copy(acc_sh.at[pl.ds(ci * K, K)],
                                acc_hbm.at[c, pl.ds(ci * K, K)])
            return 0
        lax.fori_loop(0, nrc_per_tile, dump, 0)

    fn = pl.kernel(
        body,
        out_type=jax.ShapeDtypeStruct((2, N, WID), F32),
        mesh=mesh,
        compiler_params=pltpu.CompilerParams(needs_layout_passes=False,
                                             use_tc_tiling_on_sc=False),
        scratch_types=[
            pltpu.VMEM((N,), F32),
            pltpu.VMEM((N,), F32),
            pltpu.VMEM((K,), jnp.int32),
            pltpu.VMEM((K,), jnp.int32),
            pltpu.VMEM((K,), F32),
            pltpu.VMEM((K,), jnp.int32),
            pltpu.VMEM((K,), F32),
            pltpu.VMEM((K, QW), F32),
            pltpu.VMEM((K, WID), F32),
            pltpu.VMEM_SHARED((N, WID), F32),
            pltpu.SemaphoreType.DMA,
        ],
    )
    return fn(src, dst, ae, gq, h4, asrc, adst)


def _final_body(acc0_ref, acc1_ref, acc2_ref, acc3_ref, h_ref, as_ref, ad_ref,
                q_ref, b_ref, out_ref, best_ref, brow_ref):
    i = pl.program_id(0)
    n = pl.num_programs(0)

    @pl.when(i == 0)
    def _():
        best_ref[0, 0] = jnp.float32(-3.0)

    acc0 = acc0_ref[0]
    QW = (acc0.shape[1] // 16 - 1) * 16
    rows = jnp.concatenate(
        [acc0[:, :QW], acc1_ref[0][:, :QW], acc2_ref[0][:, :QW],
         acc3_ref[0][:, :QW]], axis=1)
    denom_e = acc0[:, QW]
    sae = acc0[:, QW + 1]
    deg = acc0[:, QW + 2]
    h = h_ref[...]
    ael = sae / jnp.maximum(deg, 1.0)
    al = as_ref[0, 0] + ad_ref[0, 0] + ael
    al = jnp.maximum(al, al * 0.2)
    ploop = jnp.exp(al)
    denom = denom_e + ploop + 1e-16
    outb = (rows + ploop[:, None] * h) / denom[:, None] + b_ref[...]
    nrm = jnp.sqrt(jnp.sum(outb * outb, axis=1))
    sim = jnp.dot(outb, q_ref[0], precision=HI) / jnp.maximum(nrm, 1e-8)
    m = jnp.max(sim)
    NB = sim.shape[0]
    idxs = lax.broadcasted_iota(jnp.int32, (NB,), 0)
    loc = jnp.min(jnp.where(sim == m, idxs, NB))
    onehot = (idxs == loc).astype(F32)
    row = jnp.dot(onehot, outb, precision=HI)

    @pl.when(m > best_ref[0, 0])
    def _():
        best_ref[0, 0] = m
        brow_ref[...] = row[None, :]

    @pl.when(i == n - 1)
    def _():
        out_ref[...] = brow_ref[...]


def _final_stage(acc_a, acc_b, h, asrc, adst, q, bias, NB=1000):
    N, H = h.shape
    WID = acc_a.shape[2]
    return pl.pallas_call(
        _final_body,
        grid=(N // NB,),
        in_specs=[
            pl.BlockSpec((1, NB, WID), lambda i: (0, i, 0)),
            pl.BlockSpec((1, NB, WID), lambda i: (1, i, 0)),
            pl.BlockSpec((1, NB, WID), lambda i: (0, i, 0)),
            pl.BlockSpec((1, NB, WID), lambda i: (1, i, 0)),
            pl.BlockSpec((NB, H), lambda i: (i, 0)),
            pl.BlockSpec((1, 1, NB), lambda i: (i, 0, 0)),
            pl.BlockSpec((1, 1, NB), lambda i: (i, 0, 0)),
            pl.BlockSpec((1, H), lambda i: (0, 0)),
            pl.BlockSpec((1, H), lambda i: (0, 0)),
        ],
        out_specs=pl.BlockSpec((1, H), lambda i: (0, 0)),
        out_shape=jax.ShapeDtypeStruct((1, H), F32),
        scratch_shapes=[pltpu.SMEM((1, 1), F32), pltpu.VMEM((1, H), F32)],
    )(acc_a, acc_a, acc_b, acc_b, h, asrc, adst,
      q.reshape(1, H), bias.reshape(1, H))


def kernel(query_emb, edge_index, edge_attr, node_emb, W, att_src, att_dst,
           W_e, att_edge, bias):
    N, H = node_emb.shape
    E = edge_index.shape[1]
    h, asrc, adst = _nodes_stage(node_emb, W, att_src, att_dst)
    ae, gq = _edge_scal_stage(edge_attr, W_e, att_edge, edge_index)
    src, dst = edge_index[0], edge_index[1]
    h4 = h.reshape(4 * N, H // 4)
    ae1 = ae.reshape(E)
    as1, ad1 = asrc.reshape(N), adst.reshape(N)
    acc_a = _sc_stage(src, dst, ae1, gq, h4, as1, ad1, 0)
    acc_b = _sc_stage(src, dst, ae1, gq, h4, as1, ad1, 2)
    out = _final_stage(acc_a, acc_b, h, asrc, adst, query_emb, bias)
    return out.reshape(H)


# speedup vs baseline: 5.2042x; 1.4793x over previous
import functools

import jax
import jax.numpy as jnp
from jax import lax
from jax.experimental import pallas as pl
from jax.experimental.pallas import tpu as pltpu
from jax.experimental.pallas import tpu_sc as plsc

F32 = jnp.float32
I32 = jnp.int32
HI = lax.Precision.HIGHEST
K = 80
AW = 16


def _nodes_body(x_ref, w_ref, avs_ref, avd_ref, h_ref, aux_ref):
    x = x_ref[...]
    h = jnp.dot(x, w_ref[...], precision=HI)
    h_ref[...] = h
    asv = jnp.dot(h, avs_ref[0], precision=HI)
    adv = jnp.dot(h, avd_ref[0], precision=HI)
    col = lax.broadcasted_iota(I32, (x.shape[0], AW), 1)
    aux_ref[...] = jnp.where(col == 0, asv[:, None],
                             jnp.where(col == 1, adv[:, None], 0.0))


def _nodes_stage(node_emb, W, att_src, att_dst, NB=1000):
    N, H = node_emb.shape
    return pl.pallas_call(
        _nodes_body,
        grid=(N // NB,),
        in_specs=[
            pl.BlockSpec((NB, H), lambda i: (i, 0)),
            pl.BlockSpec((H, H), lambda i: (0, 0)),
            pl.BlockSpec((1, H), lambda i: (0, 0)),
            pl.BlockSpec((1, H), lambda i: (0, 0)),
        ],
        out_specs=[
            pl.BlockSpec((NB, H), lambda i: (i, 0)),
            pl.BlockSpec((NB, AW), lambda i: (i, 0)),
        ],
        out_shape=[
            jax.ShapeDtypeStruct((N, H), F32),
            jax.ShapeDtypeStruct((N, AW), F32),
        ],
    )(node_emb, W, att_src.reshape(1, H), att_dst.reshape(1, H))


def _edges_body(ea_ref, we_ref, av_ref, ae_ref):
    wv = jnp.dot(we_ref[...], av_ref[...], precision=HI)
    ae_ref[...] = jnp.dot(ea_ref[...], wv, precision=HI)


def _edge_scal_stage(edge_attr, W_e, att_edge, EB=1280):
    E, H = edge_attr.shape
    return pl.pallas_call(
        _edges_body,
        grid=(E // EB,),
        in_specs=[
            pl.BlockSpec((EB, H), lambda i: (i, 0)),
            pl.BlockSpec((H, H), lambda i: (0, 0)),
            pl.BlockSpec((H, 1), lambda i: (0, 0)),
        ],
        out_specs=pl.BlockSpec((EB, 1), lambda i: (i, 0)),
        out_shape=jax.ShapeDtypeStruct((E, 1), F32),
    )(edge_attr, W_e, att_edge.reshape(H, 1))


def _sc_stage(pk, h4, aux, qbase):
    nchT, threeK = pk.shape
    NV, QW = h4.shape
    N = NV // 4
    WID = QW + 16
    NSUB = 16
    nch = nchT // NSUB
    nrc = N // K
    nrc_per_tile = (nrc + NSUB - 1) // NSUB

    mesh = plsc.VectorSubcoreMesh(core_axis_name="c", subcore_axis_name="s")

    def body(pk_hbm, h4_hbm, aux_hbm, acc_hbm,
             inb, gixb, srcb, dstb, pbuf, rowbuf, scbuf, arow, drow,
             acc_sh, sem_in, sem_g0, sem_g1, sem_a0, sem_a1, sem_s0, sem_s1):
        c = lax.axis_index("c")
        s = lax.axis_index("s")
        q = qbase + c
        lanes = lax.iota(I32, 16)
        zeros16 = jnp.zeros((16,), I32)
        ones16 = zeros16 + 1
        sem_g = (sem_g0, sem_g1)
        sem_a = (sem_a0, sem_a1)
        sem_s = (sem_s0, sem_s1)

        def zrow(r, _):
            for v in range(WID // 16):
                scbuf[0, r, pl.ds(v * 16, 16)] = jnp.zeros((16,), F32)
            return 0
        lax.fori_loop(0, K, zrow, 0)

        def zchunk(k, _):
            ci = k * NSUB + s
            @pl.when(ci < nrc)
            def _():
                pltpu.sync_copy(scbuf.at[0], acc_sh.at[pl.ds(ci * K, K)])
            return 0
        lax.fori_loop(0, nrc_per_tile, zchunk, 0)
        plsc.subcore_barrier()

        def start_in(k, slot):
            pltpu.async_copy(pk_hbm.at[s * nch + k], inb.at[slot], sem_in)

        def a_phase(k, slot):
            @pl.when(k >= 2)
            def _():
                pltpu.make_async_copy(acc_hbm.at[0, pl.ds(0, K)],
                                      scbuf.at[slot], sem_s[slot]).wait()
            pltpu.make_async_copy(pk_hbm.at[0], inb.at[slot], sem_in).wait()
            for j in range(K // 16):
                d = pl.ds(j * 16, 16)
                sv = inb[slot, pl.ds(j * 16, 16)]
                dv = inb[slot, pl.ds(K + j * 16, 16)]
                av = plsc.bitcast(inb[slot, pl.ds(2 * K + j * 16, 16)], F32)
                gixb[slot, d] = sv * 4 + q
                srcb[slot, d] = sv
                dstb[slot, d] = dv
                pbuf[slot, pl.ds(K + j * 16, 16)] = av
            @pl.when(k + 1 < nch)
            def _():
                start_in(k + 1, 1 - slot)
            pltpu.async_copy(h4_hbm.at[gixb.at[slot]], rowbuf.at[slot],
                             sem_g[slot])
            pltpu.async_copy(aux_hbm.at[srcb.at[slot]], arow.at[slot],
                             sem_a[slot])
            pltpu.async_copy(aux_hbm.at[dstb.at[slot]], drow.at[slot],
                             sem_a[slot])

        def b_phase(j, slot):
            pltpu.make_async_copy(h4_hbm.at[gixb.at[slot]], rowbuf.at[slot],
                                  sem_g[slot]).wait()
            pltpu.make_async_copy(aux_hbm.at[srcb.at[slot]], arow.at[slot],
                                  sem_a[slot]).wait()
            pltpu.make_async_copy(aux_hbm.at[dstb.at[slot]], drow.at[slot],
                                  sem_a[slot]).wait()
            for j5 in range(K // 16):
                ridx = j5 * 16 + lanes
                a1 = plsc.load_gather(arow.at[slot], [ridx, zeros16])
                a2 = plsc.load_gather(drow.at[slot], [ridx, ones16])
                al = a1 + a2 + pbuf[slot, pl.ds(K + j5 * 16, 16)]
                al = jnp.maximum(al, al * 0.2)
                pbuf[slot, pl.ds(j5 * 16, 16)] = jnp.exp(al)
            def srow(e, _):
                bp = plsc.load_gather(pbuf.at[slot], [jnp.full((16,), e, I32)])
                for v in range(QW // 16):
                    scbuf[slot, e, pl.ds(v * 16, 16)] = (
                        rowbuf[slot, e, pl.ds(v * 16, 16)] * bp)
                bae = plsc.load_gather(pbuf.at[slot],
                                       [jnp.full((16,), K + e, I32)])
                extra = jnp.where(lanes == 0, bp,
                        jnp.where(lanes == 1, bae,
                        jnp.where(lanes == 2, jnp.float32(1.0),
                                  jnp.float32(0.0))))
                scbuf[slot, e, pl.ds(QW, 16)] = extra
                return 0
            lax.fori_loop(0, K, srow, 0)
            pltpu.async_copy(scbuf.at[slot], acc_sh.at[dstb.at[slot]],
                             sem_s[slot], add=True)

        start_in(0, 0)

        def pipe(k, _):
            @pl.when(k < nch)
            def _():
                @pl.when(lax.rem(k, 2) == 0)
                def _():
                    a_phase(k, 0)
                @pl.when(lax.rem(k, 2) == 1)
                def _():
                    a_phase(k, 1)
            @pl.when(k >= 1)
            def _():
                j = k - 1
                @pl.when(lax.rem(j, 2) == 0)
                def _():
                    b_phase(j, 0)
                @pl.when(lax.rem(j, 2) == 1)
                def _():
                    b_phase(j, 1)
            return 0
        lax.fori_loop(0, nch + 1, pipe, 0)

        for slot in (0, 1):
            pltpu.make_async_copy(acc_hbm.at[0, pl.ds(0, K)],
                                  scbuf.at[slot], sem_s[slot]).wait()
        plsc.subcore_barrier()

        def dump(k, _):
            ci = k * NSUB + s
            @pl.when(ci < nrc)
            def _():
                pltpu.sync_copy(acc_sh.at[pl.ds(ci * K, K)],
                                acc_hbm.at[c, pl.ds(ci * K, K)])
            return 0
        lax.fori_loop(0, nrc_per_tile, dump, 0)

    fn = pl.kernel(
        body,
        out_type=jax.ShapeDtypeStruct((2, N, WID), F32),
        mesh=mesh,
        compiler_params=pltpu.CompilerParams(needs_layout_passes=False,
                                             use_tc_tiling_on_sc=False),
        scratch_types=[
            pltpu.VMEM((2, 3 * K), I32),
            pltpu.VMEM((2, K), I32),
            pltpu.VMEM((2, K), I32),
            pltpu.VMEM((2, K), I32),
            pltpu.VMEM((2, 2 * K), F32),
            pltpu.VMEM((2, K, QW), F32),
            pltpu.VMEM((2, K, WID), F32),
            pltpu.VMEM((2, K, AW), F32),
            pltpu.VMEM((2, K, AW), F32),
            pltpu.VMEM_SHARED((N, WID), F32),
            pltpu.SemaphoreType.DMA,
            pltpu.SemaphoreType.DMA,
            pltpu.SemaphoreType.DMA,
            pltpu.SemaphoreType.DMA,
            pltpu.SemaphoreType.DMA,
            pltpu.SemaphoreType.DMA,
            pltpu.SemaphoreType.DMA,
        ],
    )
    return fn(pk, h4, aux)


def _final_body(acc0_ref, acc1_ref, acc2_ref, acc3_ref, h_ref, aux_ref,
                q_ref, b_ref, out_ref, best_ref, brow_ref):
    i = pl.program_id(0)
    n = pl.num_programs(0)

    @pl.when(i == 0)
    def _():
        best_ref[0, 0] = jnp.float32(-3.0)

    acc0 = acc0_ref[0]
    QW = (acc0.shape[1] // 16 - 1) * 16
    rows = jnp.concatenate(
        [acc0[:, :QW], acc1_ref[0][:, :QW], acc2_ref[0][:, :QW],
         acc3_ref[0][:, :QW]], axis=1)
    denom_e = acc0[:, QW]
    sae = acc0[:, QW + 1]
    deg = acc0[:, QW + 2]
    h = h_ref[...]
    ael = sae / jnp.maximum(deg, 1.0)
    al = aux_ref[:, 0] + aux_ref[:, 1] + ael
    al = jnp.maximum(al, al * 0.2)
    ploop = jnp.exp(al)
    denom = denom_e + ploop + 1e-16
    outb = (rows + ploop[:, None] * h) / denom[:, None] + b_ref[...]
    nrm = jnp.sqrt(jnp.sum(outb * outb, axis=1))
    sim = jnp.dot(outb, q_ref[0], precision=HI) / jnp.maximum(nrm, 1e-8)
    m = jnp.max(sim)
    NB = sim.shape[0]
    idxs = lax.broadcasted_iota(I32, (NB,), 0)
    loc = jnp.min(jnp.where(sim == m, idxs, NB))
    onehot = (idxs == loc).astype(F32)
    row = jnp.dot(onehot, outb, precision=HI)

    @pl.when(m > best_ref[0, 0])
    def _():
        best_ref[0, 0] = m
        brow_ref[...] = row[None, :]

    @pl.when(i == n - 1)
    def _():
        out_ref[...] = brow_ref[...]


def _final_stage(acc_a, acc_b, h, aux, q, bias, NB=1000):
    N, H = h.shape
    WID = acc_a.shape[2]
    return pl.pallas_call(
        _final_body,
        grid=(N // NB,),
        in_specs=[
            pl.BlockSpec((1, NB, WID), lambda i: (0, i, 0)),
            pl.BlockSpec((1, NB, WID), lambda i: (1, i, 0)),
            pl.BlockSpec((1, NB, WID), lambda i: (0, i, 0)),
            pl.BlockSpec((1, NB, WID), lambda i: (1, i, 0)),
            pl.BlockSpec((NB, H), lambda i: (i, 0)),
            pl.BlockSpec((NB, AW), lambda i: (i, 0)),
            pl.BlockSpec((1, H), lambda i: (0, 0)),
            pl.BlockSpec((1, H), lambda i: (0, 0)),
        ],
        out_specs=pl.BlockSpec((1, H), lambda i: (0, 0)),
        out_shape=jax.ShapeDtypeStruct((1, H), F32),
        scratch_shapes=[pltpu.SMEM((1, 1), F32), pltpu.VMEM((1, H), F32)],
    )(acc_a, acc_a, acc_b, acc_b, h, aux, q.reshape(1, H), bias.reshape(1, H))


def kernel(query_emb, edge_index, edge_attr, node_emb, W, att_src, att_dst,
           W_e, att_edge, bias):
    N, H = node_emb.shape
    E = edge_index.shape[1]
    h, aux = _nodes_stage(node_emb, W, att_src, att_dst)
    ae = _edge_scal_stage(edge_attr, W_e, att_edge)
    nch = E // K
    pk = jnp.concatenate(
        [edge_index[0].reshape(nch, K), edge_index[1].reshape(nch, K),
         lax.bitcast_convert_type(ae.reshape(nch, K), I32)], axis=1)
    h4 = h.reshape(4 * N, H // 4)
    acc_a = _sc_stage(pk, h4, aux, 0)
    acc_b = _sc_stage(pk, h4, aux, 2)
    out = _final_stage(acc_a, acc_b, h, aux, query_emb, bias)
    return out.reshape(H)
